# baseline (speedup 1.0000x reference)
import functools

import jax
import jax.numpy as jnp
from jax import lax
from jax.experimental import pallas as pl
from jax.experimental.pallas import tpu as pltpu
from jax.experimental.pallas import tpu_sc as plsc

_NUM_EXPERTS = 64
_TOP_K = 8
_HIDDEN = 4096
_TOKENS = 16384
_TB = 1024

_NW = 32
_TPW = _TOKENS // _NW
_LANES = 16


def _gate_body(x_ref, w_ref, logits_ref):
    x = x_ref[...]
    w = w_ref[...]
    logits_ref[...] = jax.lax.dot_general(
        x, w, (((1,), (1,)), ((), ())),
        preferred_element_type=jnp.float32)


def _gate_matmul(hidden_states, gate_w):
    return pl.pallas_call(
        _gate_body,
        grid=(_TOKENS // _TB,),
        in_specs=[
            pl.BlockSpec((_TB, _HIDDEN), lambda i: (i, 0)),
            pl.BlockSpec((_NUM_EXPERTS, _HIDDEN), lambda i: (0, 0)),
        ],
        out_specs=pl.BlockSpec((_TB, _NUM_EXPERTS), lambda i: (i, 0)),
        out_shape=jax.ShapeDtypeStruct((_TOKENS, _NUM_EXPERTS), jnp.float32),
        compiler_params=pltpu.CompilerParams(
            dimension_semantics=("parallel",),
        ),
    )(hidden_states, gate_w)


def _merge_desc(a, ai, b, bi):
    rb = lax.rev(b, (0,))
    rbi = lax.rev(bi, (0,))
    take = rb > a
    hv = jnp.where(take, rb, a)
    hi = jnp.where(take, rbi, ai)
    return plsc.sort_key_val(hv, hi, descending=True)


@functools.partial(
    pl.kernel,
    out_type=(
        jax.ShapeDtypeStruct((_TOKENS * _TOP_K,), jnp.float32),
        jax.ShapeDtypeStruct((_TOKENS * _TOP_K,), jnp.int32),
    ),
    mesh=plsc.VectorSubcoreMesh(core_axis_name="c", subcore_axis_name="s"),
    scratch_types=[
        pltpu.VMEM((_TPW * _NUM_EXPERTS,), jnp.float32),
        pltpu.VMEM((_TPW * _TOP_K,), jnp.float32),
        pltpu.VMEM((_TPW * _TOP_K,), jnp.int32),
    ],
    compiler_params=pltpu.CompilerParams(needs_layout_passes=False),
)
def _sc_router(logits_hbm, w_out, i_out, log_v, w_v, i_v):
    wid = lax.axis_index("s") * 2 + lax.axis_index("c")
    pltpu.sync_copy(
        logits_hbm.at[pl.ds(wid * (_TPW * _NUM_EXPERTS), _TPW * _NUM_EXPERTS)],
        log_v)

    lane = lax.broadcasted_iota(jnp.int32, (_LANES,), 0)
    mask8 = lane < _TOP_K

    def body(t, carry):
        base = t * _NUM_EXPERTS
        svs = []
        for c in range(4):
            vals = log_v[pl.ds(base + c * _LANES, _LANES)]
            idx = lane + jnp.int32(c * _LANES)
            svs.append(plsc.sort_key_val(vals, idx, descending=True))
        m01 = _merge_desc(*svs[0], *svs[1])
        m23 = _merge_desc(*svs[2], *svs[3])
        fv, fi = _merge_desc(*m01, *m23)

        m = jnp.max(fv, axis=0)
        e = jnp.where(mask8, jnp.exp(fv - m), 0.0)
        s = jnp.sum(e, axis=0)
        wts = e / s

        oidx = t * _TOP_K + jnp.where(mask8, lane, 0)
        plsc.store_scatter(w_v, [oidx], wts, mask=mask8)
        plsc.store_scatter(i_v, [oidx], jnp.where(mask8, fi, 0), mask=mask8)
        return carry

    lax.fori_loop(0, _TPW, body, 0)

    pltpu.sync_copy(w_v, w_out.at[pl.ds(wid * (_TPW * _TOP_K), _TPW * _TOP_K)])
    pltpu.sync_copy(i_v, i_out.at[pl.ds(wid * (_TPW * _TOP_K), _TPW * _TOP_K)])


def kernel(hidden_states, gate_w):
    logits = _gate_matmul(hidden_states, gate_w)
    wflat, iflat = _sc_router(logits.reshape(-1))
    return (wflat.reshape(_TOKENS, _TOP_K),
            iflat.reshape(_TOKENS, _TOP_K),
            logits)


# speedup vs baseline: 1.5640x; 1.5640x over previous
import jax
import jax.numpy as jnp
from jax.experimental import pallas as pl
from jax.experimental.pallas import tpu as pltpu

_NUM_EXPERTS = 64
_TOP_K = 8
_HIDDEN = 4096
_TOKENS = 16384
_TB = 1024


def _router_body(x_ref, w_ref, logits_ref, weights_ref, ids_ref):
    x = x_ref[...]
    w = w_ref[...]
    logits_t = jax.lax.dot_general(
        w, x, (((1,), (1,)), ((), ())),
        preferred_element_type=jnp.float32)
    logits_ref[...] = logits_t.T

    inv_row = jnp.int32(_NUM_EXPERTS - 1) - jax.lax.broadcasted_iota(
        jnp.int32, logits_t.shape, 0)
    y = jax.lax.bitcast_convert_type(logits_t, jnp.int32)
    key = y ^ (jax.lax.shift_right_arithmetic(y, 31) & jnp.int32(0x7FFFFFFF))

    neg_inf_key = jnp.int32(-2147483648)
    vals = []
    idxs = []
    for _ in range(_TOP_K):
        wmax = jnp.max(key, axis=0, keepdims=True)
        cand = jnp.where(key == wmax, inv_row, jnp.int32(-1))
        wrow = jnp.max(cand, axis=0, keepdims=True)
        idxs.append(jnp.int32(_NUM_EXPERTS - 1) - wrow)
        yb = wmax ^ (jax.lax.shift_right_arithmetic(wmax, 31)
                     & jnp.int32(0x7FFFFFFF))
        vals.append(jax.lax.bitcast_convert_type(yb, jnp.float32))
        key = jnp.where(cand == wrow, neg_inf_key, key)

    topv = jnp.concatenate(vals, axis=0)
    topi = jnp.concatenate(idxs, axis=0)
    e = jnp.exp(topv - topv[:1, :])
    wts = e / jnp.sum(e, axis=0, keepdims=True)
    weights_ref[...] = wts.T
    ids_ref[...] = topi.T


def kernel(hidden_states, gate_w):
    grid = (_TOKENS // _TB,)
    out_shape = (
        jax.ShapeDtypeStruct((_TOKENS, _NUM_EXPERTS), jnp.float32),
        jax.ShapeDtypeStruct((_TOKENS, _TOP_K), jnp.float32),
        jax.ShapeDtypeStruct((_TOKENS, _TOP_K), jnp.int32),
    )
    logits, weights, ids = pl.pallas_call(
        _router_body,
        grid=grid,
        in_specs=[
            pl.BlockSpec((_TB, _HIDDEN), lambda i: (i, 0)),
            pl.BlockSpec((_NUM_EXPERTS, _HIDDEN), lambda i: (0, 0)),
        ],
        out_specs=(
            pl.BlockSpec((_TB, _NUM_EXPERTS), lambda i: (i, 0)),
            pl.BlockSpec((_TB, _TOP_K), lambda i: (i, 0)),
            pl.BlockSpec((_TB, _TOP_K), lambda i: (i, 0)),
        ),
        out_shape=out_shape,
        compiler_params=pltpu.CompilerParams(
            dimension_semantics=("parallel",),
        ),
    )(hidden_states, gate_w)
    return weights, ids, logits
